# baseline (speedup 1.0000x reference)
import functools

import jax
import jax.numpy as jnp
from jax import lax
from jax.experimental import pallas as pl
from jax.experimental.pallas import tpu as pltpu
from jax.experimental.pallas import tpu_sc as plsc

_N = 10000
_E = 160000
_H = 256
_IN_DIM = 128
_ASSIGN = 64
_NB = 2000
_EB = 1280
_f32 = jnp.float32

_NC = 2
_NS = 16
_NW = _NC * _NS

_EH = (81920, 78080)
_EOFF = (0, 81920)


def _embed_body(h_ref, emb_ref, out_ref):
    hb = h_ref[...]
    io = lax.broadcasted_iota(jnp.int32, (_NB, _IN_DIM), 1)
    oh = (io == hb).astype(_f32)
    out_ref[...] = jnp.dot(oh, emb_ref[...], preferred_element_type=_f32)


def _embed(h_f, emb):
    return pl.pallas_call(
        _embed_body,
        grid=(_N // _NB,),
        in_specs=[
            pl.BlockSpec((_NB, 1), lambda i: (i, 0)),
            pl.BlockSpec((_IN_DIM, _H), lambda i: (0, 0)),
        ],
        out_specs=pl.BlockSpec((_NB, _H), lambda i: (i, 0)),
        out_shape=jax.ShapeDtypeStruct((_N, _H), _f32),
    )(h_f, emb)


_bf16 = jnp.bfloat16


def _nodemm_body(hf_ref, w_ref, b_ref, ah_ref, bd_ref, et_ref):
    hf = hf_ref[...]
    w = w_ref[...]
    b = b_ref[...]
    ah_ref[...] = jnp.dot(hf, w[0], preferred_element_type=_f32) + b[0:1, :]
    bh = jnp.dot(hf, w[1], preferred_element_type=_f32) + b[1:2, :]
    dh = jnp.dot(hf, w[3], preferred_element_type=_f32) + b[3:4, :]
    bd_ref[...] = jnp.concatenate([bh, dh], axis=1).astype(_bf16)
    et_ref[...] = (jnp.dot(hf, w[4], preferred_element_type=_f32)
                   + b[4:5, :]).astype(_bf16)


def _nodemm(hf, w, b):
    return pl.pallas_call(
        _nodemm_body,
        grid=(_N // _NB,),
        in_specs=[
            pl.BlockSpec((_NB, _H), lambda i: (i, 0)),
            pl.BlockSpec((5, _H, _H), lambda i: (0, 0, 0)),
            pl.BlockSpec((5, _H), lambda i: (0, 0)),
        ],
        out_specs=[
            pl.BlockSpec((_NB, _H), lambda i: (i, 0)),
            pl.BlockSpec((_NB, 2 * _H), lambda i: (i, 0)),
            pl.BlockSpec((_NB, _H), lambda i: (i, 0)),
        ],
        out_shape=[
            jax.ShapeDtypeStruct((_N, _H), _f32),
            jax.ShapeDtypeStruct((_N, 2 * _H), _bf16),
            jax.ShapeDtypeStruct((_N, _H), _bf16),
        ],
    )(hf, w, b)


def _edge_core(e_in, gbd, ge, w2, b2, ms_o):
    ce = jnp.dot(e_in, w2, preferred_element_type=_f32) + b2
    ep = gbd[:, _H:] + ge + ce
    sig = jax.nn.sigmoid(ep)
    msg = sig * gbd[:, :_H]
    ms_o[0, :, :] = msg[:, :128]
    ms_o[1, :, :] = msg[:, 128:]
    ms_o[2, :, :] = sig[:, :128]
    ms_o[3, :, :] = sig[:, 128:]
    return ep


def _acc_stats(i, ep, esum_o, esq_o):
    @pl.when(i == 0)
    def _():
        esum_o[...] = jnp.zeros_like(esum_o)
        esq_o[...] = jnp.zeros_like(esq_o)

    esum_o[...] += jnp.sum(ep, axis=0, keepdims=True)
    esq_o[...] += jnp.sum(ep * ep, axis=0, keepdims=True)


def _edge_first_body(eraw_ref, we_ref, be_ref, gbd_ref, ge_ref, w2_ref, b2_ref,
                     carry_o, epre_o, ms_o, esum_o, esq_o):
    i = pl.program_id(0)
    e_in = eraw_ref[...] * we_ref[...] + be_ref[...]
    carry_o[...] = e_in
    ep = _edge_core(e_in, gbd_ref[...].astype(_f32), ge_ref[...].astype(_f32),
                    w2_ref[...], b2_ref[...], ms_o)
    epre_o[...] = ep.astype(_bf16)
    _acc_stats(i, ep, esum_o, esq_o)


def _bn_ein(eprev_ref, carry_ref, esa_ref, esb_ref, eqa_ref, eqb_ref, bnp_ref):
    esum = esa_ref[...] + esb_ref[...]
    esq = eqa_ref[...] + eqb_ref[...]
    mean = esum * (1.0 / _E)
    var = esq * (1.0 / _E) - mean * mean
    inv = lax.rsqrt(var + 1e-5)
    g = bnp_ref[2:3, :]
    bt = bnp_ref[3:4, :]
    return carry_ref[...] + jnp.maximum(
        g * (eprev_ref[...].astype(_f32) - mean) * inv + bt, 0.0)


def _edge_mid_body(eprev_ref, carry_ref, gbd_ref, ge_ref,
                   esa_ref, esb_ref, eqa_ref, eqb_ref,
                   bnp_ref, w2_ref, b2_ref,
                   carry_o, epre_o, ms_o, esum_o, esq_o):
    i = pl.program_id(0)
    e_in = _bn_ein(eprev_ref, carry_ref, esa_ref, esb_ref, eqa_ref, eqb_ref,
                   bnp_ref)
    carry_o[...] = e_in
    ep = _edge_core(e_in, gbd_ref[...].astype(_f32), ge_ref[...].astype(_f32),
                    w2_ref[...], b2_ref[...], ms_o)
    epre_o[...] = ep.astype(_bf16)
    _acc_stats(i, ep, esum_o, esq_o)


def _edge_last_body(eprev_ref, carry_ref, gbd_ref, ge_ref,
                    esa_ref, esb_ref, eqa_ref, eqb_ref,
                    bnp_ref, w2_ref, b2_ref, ms_o):
    e_in = _bn_ein(eprev_ref, carry_ref, esa_ref, esb_ref, eqa_ref, eqb_ref,
                   bnp_ref)
    _edge_core(e_in, gbd_ref[...].astype(_f32), ge_ref[...].astype(_f32),
               w2_ref[...], b2_ref[...], ms_o)


_stat_spec = pl.BlockSpec((1, _H), lambda i: (0, 0))
_stat_shape = jax.ShapeDtypeStruct((1, _H), _f32)
_w2_spec = pl.BlockSpec((_H, _H), lambda i: (0, 0))
_erow_spec = pl.BlockSpec((_EB, _H), lambda i: (i, 0))
_ms4_spec = pl.BlockSpec((4, _EB, 128), lambda i: (0, i, 0))


def _edge_first(eraw, we, be, gbd, ge, w2, b2, eh):
    return pl.pallas_call(
        _edge_first_body,
        grid=(eh // _EB,),
        in_specs=[
            pl.BlockSpec((_EB, 1), lambda i: (i, 0)),
            _stat_spec, _stat_spec,
            pl.BlockSpec((_EB, 2 * _H), lambda i: (i, 0)),
            _erow_spec,
            _w2_spec, _stat_spec,
        ],
        out_specs=[_erow_spec, _erow_spec, _ms4_spec, _stat_spec, _stat_spec],
        out_shape=[
            jax.ShapeDtypeStruct((eh, _H), _f32),
            jax.ShapeDtypeStruct((eh, _H), _bf16),
            jax.ShapeDtypeStruct((4, eh, 128), _f32),
            _stat_shape, _stat_shape,
        ],
    )(eraw, we, be, gbd, ge, w2, b2)


def _edge_mid(eprev, carry, gbd, ge, stats, bnp, w2, b2, eh):
    return pl.pallas_call(
        _edge_mid_body,
        grid=(eh // _EB,),
        in_specs=[
            _erow_spec, _erow_spec,
            pl.BlockSpec((_EB, 2 * _H), lambda i: (i, 0)),
            _erow_spec,
            _stat_spec, _stat_spec, _stat_spec, _stat_spec,
            pl.BlockSpec((4, _H), lambda i: (0, 0)),
            _w2_spec, _stat_spec,
        ],
        out_specs=[_erow_spec, _erow_spec, _ms4_spec, _stat_spec, _stat_spec],
        out_shape=[
            jax.ShapeDtypeStruct((eh, _H), _f32),
            jax.ShapeDtypeStruct((eh, _H), _bf16),
            jax.ShapeDtypeStruct((4, eh, 128), _f32),
            _stat_shape, _stat_shape,
        ],
    )(eprev, carry, gbd, ge, *stats, bnp, w2, b2)


def _edge_last(eprev, carry, gbd, ge, stats, bnp, w2, b2, eh):
    return pl.pallas_call(
        _edge_last_body,
        grid=(eh // _EB,),
        in_specs=[
            _erow_spec, _erow_spec,
            pl.BlockSpec((_EB, 2 * _H), lambda i: (i, 0)),
            _erow_spec,
            _stat_spec, _stat_spec, _stat_spec, _stat_spec,
            pl.BlockSpec((4, _H), lambda i: (0, 0)),
            _w2_spec, _stat_spec,
        ],
        out_specs=_ms4_spec,
        out_shape=jax.ShapeDtypeStruct((4, eh, 128), _f32),
    )(eprev, carry, gbd, ge, *stats, bnp, w2, b2)


def _hnew_body(ah_ref, nd_ref, hnew_o, hsum_o, hsq_o):
    i = pl.program_id(0)
    nd = nd_ref[...]
    num = jnp.concatenate([nd[0], nd[1]], axis=1)
    den = jnp.concatenate([nd[2], nd[3]], axis=1)
    hn = ah_ref[...] + num / (den + 1e-6)
    hnew_o[...] = hn
    _acc_stats(i, hn, hsum_o, hsq_o)


def _hnew(ah, nd):
    return pl.pallas_call(
        _hnew_body,
        grid=(_N // _NB,),
        in_specs=[
            pl.BlockSpec((_NB, _H), lambda i: (i, 0)),
            pl.BlockSpec((4, _NB, 128), lambda i: (0, i, 0)),
        ],
        out_specs=[pl.BlockSpec((_NB, _H), lambda i: (i, 0)),
                   _stat_spec, _stat_spec],
        out_shape=[jax.ShapeDtypeStruct((_N, _H), _f32),
                   _stat_shape, _stat_shape],
    )(ah, nd)


def _hout_body(hin_ref, hnew_ref, hsum_ref, hsq_ref, bnp_ref, out_o):
    mean = hsum_ref[...] * (1.0 / _N)
    var = hsq_ref[...] * (1.0 / _N) - mean * mean
    inv = lax.rsqrt(var + 1e-5)
    g = bnp_ref[0:1, :]
    bt = bnp_ref[1:2, :]
    out_o[...] = hin_ref[...] + jnp.maximum(
        g * (hnew_ref[...] - mean) * inv + bt, 0.0)


def _hout(hin, hnew, hsum, hsq, bnp):
    return pl.pallas_call(
        _hout_body,
        grid=(_N // _NB,),
        in_specs=[
            pl.BlockSpec((_NB, _H), lambda i: (i, 0)),
            pl.BlockSpec((_NB, _H), lambda i: (i, 0)),
            _stat_spec, _stat_spec,
            pl.BlockSpec((4, _H), lambda i: (0, 0)),
        ],
        out_specs=pl.BlockSpec((_NB, _H), lambda i: (i, 0)),
        out_shape=jax.ShapeDtypeStruct((_N, _H), _f32),
    )(hin, hnew, hsum, hsq, bnp)


def _assign_body(hf_ref, ws_ref, bs_ref, out_o):
    lg = jnp.dot(hf_ref[...], ws_ref[...], preferred_element_type=_f32) + bs_ref[...]
    m = jnp.max(lg, axis=1, keepdims=True)
    ex = jnp.exp(lg - m)
    out_o[...] = ex / jnp.sum(ex, axis=1, keepdims=True)


def _assign(hf, ws, bs):
    return pl.pallas_call(
        _assign_body,
        grid=(_N // _NB,),
        in_specs=[
            pl.BlockSpec((_NB, _H), lambda i: (i, 0)),
            pl.BlockSpec((_H, _ASSIGN), lambda i: (0, 0)),
            pl.BlockSpec((1, _ASSIGN), lambda i: (0, 0)),
        ],
        out_specs=pl.BlockSpec((_NB, _ASSIGN), lambda i: (i, 0)),
        out_shape=jax.ShapeDtypeStruct((_N, _ASSIGN), _f32),
    )(hf, ws, bs)


def _readout_body(hf_ref, w1_ref, b1_ref, w2_ref, b2_ref, w3_ref, b3_ref, out_o):
    x = jnp.maximum(
        jnp.dot(hf_ref[...], w1_ref[...], preferred_element_type=_f32) + b1_ref[...], 0.0)
    x = jnp.maximum(
        jnp.dot(x, w2_ref[...], preferred_element_type=_f32) + b2_ref[...], 0.0)
    out_o[...] = jnp.dot(x, w3_ref[...], preferred_element_type=_f32) + b3_ref[...]


def _readout(hf, w1, b1, w2, b2, w3, b3):
    return pl.pallas_call(
        _readout_body,
        grid=(_N // _NB,),
        in_specs=[
            pl.BlockSpec((_NB, _H), lambda i: (i, 0)),
            pl.BlockSpec((_H, _H // 2), lambda i: (0, 0)),
            pl.BlockSpec((1, _H // 2), lambda i: (0, 0)),
            pl.BlockSpec((_H // 2, _H // 4), lambda i: (0, 0)),
            pl.BlockSpec((1, _H // 4), lambda i: (0, 0)),
            pl.BlockSpec((_H // 4, 8), lambda i: (0, 0)),
            pl.BlockSpec((1, 8), lambda i: (0, 0)),
        ],
        out_specs=pl.BlockSpec((_NB, 8), lambda i: (i, 0)),
        out_shape=jax.ShapeDtypeStruct((_N, 8), _f32),
    )(hf, w1, b1, w2, b2, w3, b3)


_NROW = 624
_NREM = _N - _NS * _NROW


def _mesh():
    return plsc.VectorSubcoreMesh(
        core_axis_name="c", subcore_axis_name="s",
        num_cores=_NC, num_subcores=_NS)


def _gather_sc(bd, et, src, dst, eh, ch, slots):
    wbd = _H
    wet = _H // 2
    per = eh // _NW
    nch = per // ch
    rem = per - nch * ch
    full = (nch // slots) * slots
    nit = full // slots

    @functools.partial(
        pl.kernel,
        out_type=[
            jax.ShapeDtypeStruct((eh, wbd), jnp.int32),
            jax.ShapeDtypeStruct((eh, wet), jnp.int32),
        ],
        mesh=_mesh(),
        scratch_types=[
            pltpu.VMEM((per,), jnp.int32),
            pltpu.VMEM((per,), jnp.int32),
            [pltpu.VMEM((ch, wbd), jnp.int32) for _ in range(slots)],
            [pltpu.VMEM((ch, wet), jnp.int32) for _ in range(slots)],
            [pltpu.SemaphoreType.DMA for _ in range(slots)],
            [pltpu.SemaphoreType.DMA for _ in range(slots)],
        ],
    )
    def k(bd_hbm, et_hbm, src_hbm, dst_hbm, gbd_hbm, ge_hbm,
          idx_s, idx_d, bd_bufs, e_bufs, gsems, wsems):
        c = lax.axis_index("c")
        s = lax.axis_index("s")
        wid = s * _NC + c
        start = wid * per
        pltpu.sync_copy(src_hbm.at[pl.ds(start, per)], idx_s)
        pltpu.sync_copy(dst_hbm.at[pl.ds(start, per)], idx_d)

        def g_start(chk, b):
            off = chk * ch
            pltpu.async_copy(bd_hbm.at[idx_s.at[pl.ds(off, ch)]],
                             bd_bufs[b], gsems[b])
            pltpu.async_copy(et_hbm.at[idx_d.at[pl.ds(off, ch)]],
                             e_bufs[b], gsems[b])

        def g_wait(b):
            pltpu.make_async_copy(bd_hbm.at[idx_s.at[pl.ds(0, ch)]],
                                  bd_bufs[b], gsems[b]).wait()
            pltpu.make_async_copy(et_hbm.at[idx_d.at[pl.ds(0, ch)]],
                                  e_bufs[b], gsems[b]).wait()

        def w_start(chk, b):
            off = start + chk * ch
            pltpu.async_copy(bd_bufs[b], gbd_hbm.at[pl.ds(off, ch)], wsems[b])
            pltpu.async_copy(e_bufs[b], ge_hbm.at[pl.ds(off, ch)], wsems[b])

        def w_wait(b):
            pltpu.make_async_copy(bd_bufs[b], gbd_hbm.at[pl.ds(0, ch)],
                                  wsems[b]).wait()
            pltpu.make_async_copy(e_bufs[b], ge_hbm.at[pl.ds(0, ch)],
                                  wsems[b]).wait()

        for b in range(slots):
            g_start(b, b)

        def body(i2, carry):
            for b in range(slots):
                chk = i2 * slots + b
                g_wait(b)
                w_start(chk, b)
                nxt = chk + slots

                @pl.when(nxt < nch)
                def _():
                    w_wait(b)
                    g_start(nxt, b)

            return carry

        lax.fori_loop(0, nit, body, 0)
        for t in range(full, nch):
            b = t % slots
            g_wait(b)
            w_start(t, b)
        for b in range(slots):
            w_wait(b)
        if rem:
            toff = nch * ch
            base = start + toff
            pltpu.sync_copy(bd_hbm.at[idx_s.at[pl.ds(toff, rem)]],
                            bd_bufs[0].at[pl.ds(0, rem)])
            pltpu.sync_copy(et_hbm.at[idx_d.at[pl.ds(toff, rem)]],
                            e_bufs[0].at[pl.ds(0, rem)])
            pltpu.sync_copy(bd_bufs[0].at[pl.ds(0, rem)],
                            gbd_hbm.at[pl.ds(base, rem)])
            pltpu.sync_copy(e_bufs[0].at[pl.ds(0, rem)],
                            ge_hbm.at[pl.ds(base, rem)])

    return k(bd, et, src, dst)


def _scatter_sc(ms4, dst, init, eh, ch, slots):
    ms_flat = ms4.reshape(4 * eh, 128)
    per = eh // _NS
    nch = per // ch
    full = (nch // slots) * slots
    nit = full // slots

    @functools.partial(
        pl.kernel,
        out_type=jax.ShapeDtypeStruct((4 * _N, 128), _f32),
        mesh=_mesh(),
        scratch_types=[
            pltpu.VMEM_SHARED((_N, 128), _f32),
            [pltpu.VMEM((ch,), jnp.int32) for _ in range(slots)],
            [pltpu.VMEM((ch, 128), _f32) for _ in range(slots)],
            [pltpu.SemaphoreType.DMA for _ in range(slots)],
            [pltpu.SemaphoreType.DMA for _ in range(slots)],
        ],
    )
    def k(ms_hbm, dst_hbm, init_hbm, out_hbm, accum, idx_bufs, ms_bufs,
          lsems, asems):
        c = lax.axis_index("c")
        s = lax.axis_index("s")
        rem0 = _NS * _NROW

        def l_start(q, chk, b):
            base = s * per + chk * ch
            pltpu.async_copy(dst_hbm.at[pl.ds(base, ch)], idx_bufs[b],
                             lsems[b])
            pltpu.async_copy(ms_hbm.at[pl.ds(q * eh + base, ch)],
                             ms_bufs[b], lsems[b])

        def l_wait(b):
            pltpu.make_async_copy(dst_hbm.at[pl.ds(0, ch)], idx_bufs[b],
                                  lsems[b]).wait()
            pltpu.make_async_copy(ms_hbm.at[pl.ds(0, ch)], ms_bufs[b],
                                  lsems[b]).wait()

        def a_start(b):
            pltpu.async_copy(ms_bufs[b], accum.at[idx_bufs[b]], asems[b],
                             add=True)

        def a_wait(b):
            pltpu.make_async_copy(ms_bufs[b], accum.at[idx_bufs[b]],
                                  asems[b]).wait()

        for phase in range(2):
            q = c * 2 + phase
            pltpu.sync_copy(init_hbm.at[pl.ds(q * _N + s * _NROW, _NROW)],
                            accum.at[pl.ds(s * _NROW, _NROW)])

            @pl.when(s == _NS - 1)
            def _():
                pltpu.sync_copy(init_hbm.at[pl.ds(q * _N + rem0, _NREM)],
                                accum.at[pl.ds(rem0, _NREM)])

            plsc.subcore_barrier()

            for b in range(slots):
                l_start(q, b, b)

            def body(i2, carry):
                for b in range(slots):
                    chk = i2 * slots + b
                    l_wait(b)
                    a_start(b)
                    nxt = chk + slots

                    @pl.when(nxt < nch)
                    def _():
                        a_wait(b)
                        l_start(q, nxt, b)

                return carry

            lax.fori_loop(0, nit, body, 0)
            for t in range(full, nch):
                b = t % slots
                l_wait(b)
                a_start(b)
            for b in range(slots):
                a_wait(b)
            plsc.subcore_barrier()
            pltpu.sync_copy(accum.at[pl.ds(s * _NROW, _NROW)],
                            out_hbm.at[pl.ds(q * _N + s * _NROW, _NROW)])

            @pl.when(s == _NS - 1)
            def _():
                pltpu.sync_copy(accum.at[pl.ds(rem0, _NREM)],
                                out_hbm.at[pl.ds(q * _N + rem0, _NREM)])

            plsc.subcore_barrier()

    return k(ms_flat, dst, init)


def kernel(h, edge_index, e, emb_h, We, be, Wl, bl, bn, Ws, bs,
           W1, b1, W2, b2, W3, b3):
    src = [lax.slice_in_dim(edge_index[0], _EOFF[j], _EOFF[j] + _EH[j])
           for j in range(2)]
    dst = [lax.slice_in_dim(edge_index[1], _EOFF[j], _EOFF[j] + _EH[j])
           for j in range(2)]
    eraw = [lax.slice_in_dim(e, _EOFF[j], _EOFF[j] + _EH[j]) for j in range(2)]
    h_f = h.reshape(_N, 1)
    we2 = We.reshape(1, _H)
    be2 = be.reshape(1, _H)
    zeros4n = jnp.zeros((4 * _N, 128), _f32)

    hf = _embed(h_f, emb_h)
    eprev = [None, None]
    carry = [None, None]
    stats = [None, None]
    s0 = None
    for i in range(4):
        ah, bd_t, et_t = _nodemm(hf, Wl[i], bl[i])
        bd_i = lax.bitcast_convert_type(bd_t.reshape(_N, _H, 2), jnp.int32)
        et_i = lax.bitcast_convert_type(et_t.reshape(_N, _H // 2, 2), jnp.int32)
        w2 = Wl[i, 2]
        b2e = bl[i, 2].reshape(1, _H)
        gath = []
        for j in range(2):
            gbd_i, ge_i = _gather_sc(bd_i, et_i, src[j], dst[j], _EH[j], 32, 4)
            gath.append((
                lax.bitcast_convert_type(gbd_i, _bf16).reshape(_EH[j], 2 * _H),
                lax.bitcast_convert_type(ge_i, _bf16).reshape(_EH[j], _H),
            ))
        ms4 = [None, None]
        if i == 0:
            for j in range(2):
                carry[j], eprev[j], ms4[j], es, eq = _edge_first(
                    eraw[j], we2, be2, gath[j][0], gath[j][1], w2, b2e,
                    _EH[j])
                stats[j] = (es, eq)
        else:
            allstats = (stats[0][0], stats[1][0], stats[0][1], stats[1][1])
            if i < 3:
                nstats = [None, None]
                for j in range(2):
                    carry[j], eprev[j], ms4[j], es, eq = _edge_mid(
                        eprev[j], carry[j], gath[j][0], gath[j][1],
                        allstats, bn[i - 1], w2, b2e, _EH[j])
                    nstats[j] = (es, eq)
                stats = nstats
            else:
                for j in range(2):
                    ms4[j] = _edge_last(
                        eprev[j], carry[j], gath[j][0], gath[j][1],
                        allstats, bn[i - 1], w2, b2e, _EH[j])
        nd = _scatter_sc(ms4[0], dst[0], zeros4n, _EH[0], 80, 4)
        nd = _scatter_sc(ms4[1], dst[1], nd, _EH[1], 80, 4)
        hnew, hsum, hsq = _hnew(ah, nd.reshape(4, _N, 128))
        hf = _hout(hf, hnew, hsum, hsq, bn[i])
        if i == 2:
            s0 = _assign(hf, Ws, bs.reshape(1, _ASSIGN))

    h_out = _readout(hf, W1, b1.reshape(1, _H // 2),
                     W2, b2.reshape(1, _H // 4),
                     W3, b3.reshape(1, 8))
    return (h_out, s0.reshape(1, _N, _ASSIGN))


# speedup vs baseline: 4.2026x; 4.2026x over previous
import functools

import jax
import jax.numpy as jnp
from jax import lax
from jax.experimental import pallas as pl
from jax.experimental.pallas import tpu as pltpu
from jax.experimental.pallas import tpu_sc as plsc

_N = 10000
_E = 160000
_H = 256
_IN_DIM = 128
_ASSIGN = 64
_NB = 2000
_EB = 1280
_f32 = jnp.float32

_NC = 2
_NS = 16
_NW = _NC * _NS

_EH = (81920, 78080)
_EOFF = (0, 81920)


def _embed_body(h_ref, emb_ref, out_ref):
    hb = h_ref[...]
    io = lax.broadcasted_iota(jnp.int32, (_NB, _IN_DIM), 1)
    oh = (io == hb).astype(_f32)
    out_ref[...] = jnp.dot(oh, emb_ref[...], preferred_element_type=_f32)


def _embed(h_f, emb):
    return pl.pallas_call(
        _embed_body,
        grid=(_N // _NB,),
        in_specs=[
            pl.BlockSpec((_NB, 1), lambda i: (i, 0)),
            pl.BlockSpec((_IN_DIM, _H), lambda i: (0, 0)),
        ],
        out_specs=pl.BlockSpec((_NB, _H), lambda i: (i, 0)),
        out_shape=jax.ShapeDtypeStruct((_N, _H), _f32),
    )(h_f, emb)


_HI_MASK = -65536


def _pack2(lo, hi):
    lob = lax.bitcast_convert_type(lo, jnp.int32) + 0x8000
    hib = lax.bitcast_convert_type(hi, jnp.int32) + 0x8000
    return jnp.bitwise_or(jnp.bitwise_and(hib, _HI_MASK),
                          jnp.bitwise_and(jnp.right_shift(lob, 16), 0xFFFF))


def _unpack_lo(w):
    return lax.bitcast_convert_type(jnp.left_shift(w, 16), _f32)


def _unpack_hi(w):
    return lax.bitcast_convert_type(jnp.bitwise_and(w, _HI_MASK), _f32)


def _nodemm_body(hf_ref, w_ref, b_ref, ah_ref, bd_ref, et_ref):
    hf = hf_ref[...]
    w = w_ref[...]
    b = b_ref[...]
    ah_ref[...] = jnp.dot(hf, w[0], preferred_element_type=_f32) + b[0:1, :]
    bh = jnp.dot(hf, w[1], preferred_element_type=_f32) + b[1:2, :]
    dh = jnp.dot(hf, w[3], preferred_element_type=_f32) + b[3:4, :]
    bd_ref[...] = _pack2(bh, dh)
    eh = jnp.dot(hf, w[4], preferred_element_type=_f32) + b[4:5, :]
    et_ref[...] = _pack2(eh[:, :_H // 2], eh[:, _H // 2:])


def _nodemm(hf, w, b):
    return pl.pallas_call(
        _nodemm_body,
        grid=(_N // _NB,),
        in_specs=[
            pl.BlockSpec((_NB, _H), lambda i: (i, 0)),
            pl.BlockSpec((5, _H, _H), lambda i: (0, 0, 0)),
            pl.BlockSpec((5, _H), lambda i: (0, 0)),
        ],
        out_specs=[
            pl.BlockSpec((_NB, _H), lambda i: (i, 0)),
            pl.BlockSpec((_NB, _H), lambda i: (i, 0)),
            pl.BlockSpec((_NB, _H // 2), lambda i: (i, 0)),
        ],
        out_shape=[
            jax.ShapeDtypeStruct((_N, _H), _f32),
            jax.ShapeDtypeStruct((_N, _H), jnp.int32),
            jax.ShapeDtypeStruct((_N, _H // 2), jnp.int32),
        ],
    )(hf, w, b)


def _edge_core(e_in, gbd_w, ge_w, w2, b2, ms_o):
    ce = jnp.dot(e_in, w2, preferred_element_type=_f32) + b2
    ge = jnp.concatenate([_unpack_lo(ge_w), _unpack_hi(ge_w)], axis=1)
    ep = _unpack_hi(gbd_w) + ge + ce
    sig = jax.nn.sigmoid(ep)
    msg = sig * _unpack_lo(gbd_w)
    ms_o[0, :, :] = msg[:, :128]
    ms_o[1, :, :] = msg[:, 128:]
    ms_o[2, :, :] = sig[:, :128]
    ms_o[3, :, :] = sig[:, 128:]
    return ep


def _acc_stats(i, ep, esum_o, esq_o):
    @pl.when(i == 0)
    def _():
        esum_o[...] = jnp.zeros_like(esum_o)
        esq_o[...] = jnp.zeros_like(esq_o)

    esum_o[...] += jnp.sum(ep, axis=0, keepdims=True)
    esq_o[...] += jnp.sum(ep * ep, axis=0, keepdims=True)


def _edge_first_body(eraw_ref, we_ref, be_ref, gbd_ref, ge_ref, w2_ref, b2_ref,
                     carry_o, epre_o, ms_o, esum_o, esq_o):
    i = pl.program_id(0)
    e_in = eraw_ref[...] * we_ref[...] + be_ref[...]
    carry_o[...] = e_in
    ep = _edge_core(e_in, gbd_ref[...], ge_ref[...],
                    w2_ref[...], b2_ref[...], ms_o)
    epre_o[...] = _pack2(ep[:, :_H // 2], ep[:, _H // 2:])
    _acc_stats(i, ep, esum_o, esq_o)


def _bn_ein(eprev_ref, carry_ref, esa_ref, esb_ref, eqa_ref, eqb_ref, bnp_ref):
    esum = esa_ref[...] + esb_ref[...]
    esq = eqa_ref[...] + eqb_ref[...]
    mean = esum * (1.0 / _E)
    var = esq * (1.0 / _E) - mean * mean
    inv = lax.rsqrt(var + 1e-5)
    g = bnp_ref[2:3, :]
    bt = bnp_ref[3:4, :]
    ew = eprev_ref[...]
    ep = jnp.concatenate([_unpack_lo(ew), _unpack_hi(ew)], axis=1)
    return carry_ref[...] + jnp.maximum(g * (ep - mean) * inv + bt, 0.0)


def _edge_mid_body(eprev_ref, carry_ref, gbd_ref, ge_ref,
                   esa_ref, esb_ref, eqa_ref, eqb_ref,
                   bnp_ref, w2_ref, b2_ref,
                   carry_o, epre_o, ms_o, esum_o, esq_o):
    i = pl.program_id(0)
    e_in = _bn_ein(eprev_ref, carry_ref, esa_ref, esb_ref, eqa_ref, eqb_ref,
                   bnp_ref)
    carry_o[...] = e_in
    ep = _edge_core(e_in, gbd_ref[...], ge_ref[...],
                    w2_ref[...], b2_ref[...], ms_o)
    epre_o[...] = _pack2(ep[:, :_H // 2], ep[:, _H // 2:])
    _acc_stats(i, ep, esum_o, esq_o)


def _edge_last_body(eprev_ref, carry_ref, gbd_ref, ge_ref,
                    esa_ref, esb_ref, eqa_ref, eqb_ref,
                    bnp_ref, w2_ref, b2_ref, ms_o):
    e_in = _bn_ein(eprev_ref, carry_ref, esa_ref, esb_ref, eqa_ref, eqb_ref,
                   bnp_ref)
    _edge_core(e_in, gbd_ref[...], ge_ref[...],
               w2_ref[...], b2_ref[...], ms_o)


_stat_spec = pl.BlockSpec((1, _H), lambda i: (0, 0))
_stat_shape = jax.ShapeDtypeStruct((1, _H), _f32)
_w2_spec = pl.BlockSpec((_H, _H), lambda i: (0, 0))
_erow_spec = pl.BlockSpec((_EB, _H), lambda i: (i, 0))
_ehw_spec = pl.BlockSpec((_EB, _H // 2), lambda i: (i, 0))
_ms4_spec = pl.BlockSpec((4, _EB, 128), lambda i: (0, i, 0))


def _edge_first(eraw, we, be, gbd, ge, w2, b2, eh):
    return pl.pallas_call(
        _edge_first_body,
        grid=(eh // _EB,),
        in_specs=[
            pl.BlockSpec((_EB, 1), lambda i: (i, 0)),
            _stat_spec, _stat_spec,
            _erow_spec,
            _ehw_spec,
            _w2_spec, _stat_spec,
        ],
        out_specs=[_erow_spec, _ehw_spec, _ms4_spec, _stat_spec, _stat_spec],
        out_shape=[
            jax.ShapeDtypeStruct((eh, _H), _f32),
            jax.ShapeDtypeStruct((eh, _H // 2), jnp.int32),
            jax.ShapeDtypeStruct((4, eh, 128), _f32),
            _stat_shape, _stat_shape,
        ],
    )(eraw, we, be, gbd, ge, w2, b2)


def _edge_mid(eprev, carry, gbd, ge, stats, bnp, w2, b2, eh):
    return pl.pallas_call(
        _edge_mid_body,
        grid=(eh // _EB,),
        in_specs=[
            _ehw_spec, _erow_spec,
            _erow_spec,
            _ehw_spec,
            _stat_spec, _stat_spec, _stat_spec, _stat_spec,
            pl.BlockSpec((4, _H), lambda i: (0, 0)),
            _w2_spec, _stat_spec,
        ],
        out_specs=[_erow_spec, _ehw_spec, _ms4_spec, _stat_spec, _stat_spec],
        out_shape=[
            jax.ShapeDtypeStruct((eh, _H), _f32),
            jax.ShapeDtypeStruct((eh, _H // 2), jnp.int32),
            jax.ShapeDtypeStruct((4, eh, 128), _f32),
            _stat_shape, _stat_shape,
        ],
    )(eprev, carry, gbd, ge, *stats, bnp, w2, b2)


def _edge_last(eprev, carry, gbd, ge, stats, bnp, w2, b2, eh):
    return pl.pallas_call(
        _edge_last_body,
        grid=(eh // _EB,),
        in_specs=[
            _ehw_spec, _erow_spec,
            _erow_spec,
            _ehw_spec,
            _stat_spec, _stat_spec, _stat_spec, _stat_spec,
            pl.BlockSpec((4, _H), lambda i: (0, 0)),
            _w2_spec, _stat_spec,
        ],
        out_specs=_ms4_spec,
        out_shape=jax.ShapeDtypeStruct((4, eh, 128), _f32),
    )(eprev, carry, gbd, ge, *stats, bnp, w2, b2)


def _hnew_body(ah_ref, nd_ref, hnew_o, hsum_o, hsq_o):
    i = pl.program_id(0)
    nd = nd_ref[...]
    num = jnp.concatenate([nd[0], nd[1]], axis=1)
    den = jnp.concatenate([nd[2], nd[3]], axis=1)
    hn = ah_ref[...] + num / (den + 1e-6)
    hnew_o[...] = hn
    _acc_stats(i, hn, hsum_o, hsq_o)


def _hnew(ah, nd):
    return pl.pallas_call(
        _hnew_body,
        grid=(_N // _NB,),
        in_specs=[
            pl.BlockSpec((_NB, _H), lambda i: (i, 0)),
            pl.BlockSpec((4, _NB, 128), lambda i: (0, i, 0)),
        ],
        out_specs=[pl.BlockSpec((_NB, _H), lambda i: (i, 0)),
                   _stat_spec, _stat_spec],
        out_shape=[jax.ShapeDtypeStruct((_N, _H), _f32),
                   _stat_shape, _stat_shape],
    )(ah, nd)


def _hout_body(hin_ref, hnew_ref, hsum_ref, hsq_ref, bnp_ref, out_o):
    mean = hsum_ref[...] * (1.0 / _N)
    var = hsq_ref[...] * (1.0 / _N) - mean * mean
    inv = lax.rsqrt(var + 1e-5)
    g = bnp_ref[0:1, :]
    bt = bnp_ref[1:2, :]
    out_o[...] = hin_ref[...] + jnp.maximum(
        g * (hnew_ref[...] - mean) * inv + bt, 0.0)


def _hout(hin, hnew, hsum, hsq, bnp):
    return pl.pallas_call(
        _hout_body,
        grid=(_N // _NB,),
        in_specs=[
            pl.BlockSpec((_NB, _H), lambda i: (i, 0)),
            pl.BlockSpec((_NB, _H), lambda i: (i, 0)),
            _stat_spec, _stat_spec,
            pl.BlockSpec((4, _H), lambda i: (0, 0)),
        ],
        out_specs=pl.BlockSpec((_NB, _H), lambda i: (i, 0)),
        out_shape=jax.ShapeDtypeStruct((_N, _H), _f32),
    )(hin, hnew, hsum, hsq, bnp)


def _assign_body(hf_ref, ws_ref, bs_ref, out_o):
    lg = jnp.dot(hf_ref[...], ws_ref[...], preferred_element_type=_f32) + bs_ref[...]
    m = jnp.max(lg, axis=1, keepdims=True)
    ex = jnp.exp(lg - m)
    out_o[...] = ex / jnp.sum(ex, axis=1, keepdims=True)


def _assign(hf, ws, bs):
    return pl.pallas_call(
        _assign_body,
        grid=(_N // _NB,),
        in_specs=[
            pl.BlockSpec((_NB, _H), lambda i: (i, 0)),
            pl.BlockSpec((_H, _ASSIGN), lambda i: (0, 0)),
            pl.BlockSpec((1, _ASSIGN), lambda i: (0, 0)),
        ],
        out_specs=pl.BlockSpec((_NB, _ASSIGN), lambda i: (i, 0)),
        out_shape=jax.ShapeDtypeStruct((_N, _ASSIGN), _f32),
    )(hf, ws, bs)


def _readout_body(hf_ref, w1_ref, b1_ref, w2_ref, b2_ref, w3_ref, b3_ref, out_o):
    x = jnp.maximum(
        jnp.dot(hf_ref[...], w1_ref[...], preferred_element_type=_f32) + b1_ref[...], 0.0)
    x = jnp.maximum(
        jnp.dot(x, w2_ref[...], preferred_element_type=_f32) + b2_ref[...], 0.0)
    out_o[...] = jnp.dot(x, w3_ref[...], preferred_element_type=_f32) + b3_ref[...]


def _readout(hf, w1, b1, w2, b2, w3, b3):
    return pl.pallas_call(
        _readout_body,
        grid=(_N // _NB,),
        in_specs=[
            pl.BlockSpec((_NB, _H), lambda i: (i, 0)),
            pl.BlockSpec((_H, _H // 2), lambda i: (0, 0)),
            pl.BlockSpec((1, _H // 2), lambda i: (0, 0)),
            pl.BlockSpec((_H // 2, _H // 4), lambda i: (0, 0)),
            pl.BlockSpec((1, _H // 4), lambda i: (0, 0)),
            pl.BlockSpec((_H // 4, 8), lambda i: (0, 0)),
            pl.BlockSpec((1, 8), lambda i: (0, 0)),
        ],
        out_specs=pl.BlockSpec((_NB, 8), lambda i: (i, 0)),
        out_shape=jax.ShapeDtypeStruct((_N, 8), _f32),
    )(hf, w1, b1, w2, b2, w3, b3)


_NROW = 624
_NREM = _N - _NS * _NROW


def _mesh():
    return plsc.VectorSubcoreMesh(
        core_axis_name="c", subcore_axis_name="s",
        num_cores=_NC, num_subcores=_NS)


def _gather_sc(bd, et, src, dst, eh, ch, slots):
    wbd = _H
    wet = _H // 2
    per = eh // _NW
    nch = per // ch
    rem = per - nch * ch
    full = (nch // slots) * slots
    nit = full // slots

    @functools.partial(
        pl.kernel,
        out_type=[
            jax.ShapeDtypeStruct((eh, wbd), jnp.int32),
            jax.ShapeDtypeStruct((eh, wet), jnp.int32),
        ],
        mesh=_mesh(),
        scratch_types=[
            pltpu.VMEM((per,), jnp.int32),
            pltpu.VMEM((per,), jnp.int32),
            [pltpu.VMEM((ch, wbd), jnp.int32) for _ in range(slots)],
            [pltpu.VMEM((ch, wet), jnp.int32) for _ in range(slots)],
            [pltpu.SemaphoreType.DMA for _ in range(slots)],
            [pltpu.SemaphoreType.DMA for _ in range(slots)],
        ],
    )
    def k(bd_hbm, et_hbm, src_hbm, dst_hbm, gbd_hbm, ge_hbm,
          idx_s, idx_d, bd_bufs, e_bufs, gsems, wsems):
        c = lax.axis_index("c")
        s = lax.axis_index("s")
        wid = s * _NC + c
        start = wid * per
        pltpu.sync_copy(src_hbm.at[pl.ds(start, per)], idx_s)
        pltpu.sync_copy(dst_hbm.at[pl.ds(start, per)], idx_d)

        def g_start(chk, b):
            off = chk * ch
            pltpu.async_copy(bd_hbm.at[idx_s.at[pl.ds(off, ch)]],
                             bd_bufs[b], gsems[b])
            pltpu.async_copy(et_hbm.at[idx_d.at[pl.ds(off, ch)]],
                             e_bufs[b], gsems[b])

        def g_wait(b):
            pltpu.make_async_copy(bd_hbm.at[idx_s.at[pl.ds(0, ch)]],
                                  bd_bufs[b], gsems[b]).wait()
            pltpu.make_async_copy(et_hbm.at[idx_d.at[pl.ds(0, ch)]],
                                  e_bufs[b], gsems[b]).wait()

        def w_start(chk, b):
            off = start + chk * ch
            pltpu.async_copy(bd_bufs[b], gbd_hbm.at[pl.ds(off, ch)], wsems[b])
            pltpu.async_copy(e_bufs[b], ge_hbm.at[pl.ds(off, ch)], wsems[b])

        def w_wait(b):
            pltpu.make_async_copy(bd_bufs[b], gbd_hbm.at[pl.ds(0, ch)],
                                  wsems[b]).wait()
            pltpu.make_async_copy(e_bufs[b], ge_hbm.at[pl.ds(0, ch)],
                                  wsems[b]).wait()

        for b in range(slots):
            g_start(b, b)

        def body(i2, carry):
            for b in range(slots):
                chk = i2 * slots + b
                g_wait(b)
                w_start(chk, b)
                nxt = chk + slots

                @pl.when(nxt < nch)
                def _():
                    w_wait(b)
                    g_start(nxt, b)

            return carry

        lax.fori_loop(0, nit, body, 0)
        for t in range(full, nch):
            b = t % slots
            g_wait(b)
            w_start(t, b)
        for b in range(slots):
            w_wait(b)
        if rem:
            toff = nch * ch
            base = start + toff
            pltpu.sync_copy(bd_hbm.at[idx_s.at[pl.ds(toff, rem)]],
                            bd_bufs[0].at[pl.ds(0, rem)])
            pltpu.sync_copy(et_hbm.at[idx_d.at[pl.ds(toff, rem)]],
                            e_bufs[0].at[pl.ds(0, rem)])
            pltpu.sync_copy(bd_bufs[0].at[pl.ds(0, rem)],
                            gbd_hbm.at[pl.ds(base, rem)])
            pltpu.sync_copy(e_bufs[0].at[pl.ds(0, rem)],
                            ge_hbm.at[pl.ds(base, rem)])

    return k(bd, et, src, dst)


def _scatter_sc(ms4, dst, init, eh, ch, slots):
    ms_flat = ms4.reshape(4 * eh, 128)
    per = eh // _NS
    nch = per // ch
    full = (nch // slots) * slots
    nit = full // slots

    @functools.partial(
        pl.kernel,
        out_type=jax.ShapeDtypeStruct((4 * _N, 128), _f32),
        mesh=_mesh(),
        scratch_types=[
            pltpu.VMEM_SHARED((_N, 128), _f32),
            [pltpu.VMEM((ch,), jnp.int32) for _ in range(slots)],
            [pltpu.VMEM((ch, 128), _f32) for _ in range(slots)],
            [pltpu.SemaphoreType.DMA for _ in range(slots)],
            [pltpu.SemaphoreType.DMA for _ in range(slots)],
        ],
    )
    def k(ms_hbm, dst_hbm, init_hbm, out_hbm, accum, idx_bufs, ms_bufs,
          lsems, asems):
        c = lax.axis_index("c")
        s = lax.axis_index("s")
        rem0 = _NS * _NROW

        def l_start(q, chk, b):
            base = s * per + chk * ch
            pltpu.async_copy(dst_hbm.at[pl.ds(base, ch)], idx_bufs[b],
                             lsems[b])
            pltpu.async_copy(ms_hbm.at[pl.ds(q * eh + base, ch)],
                             ms_bufs[b], lsems[b])

        def l_wait(b):
            pltpu.make_async_copy(dst_hbm.at[pl.ds(0, ch)], idx_bufs[b],
                                  lsems[b]).wait()
            pltpu.make_async_copy(ms_hbm.at[pl.ds(0, ch)], ms_bufs[b],
                                  lsems[b]).wait()

        def a_start(b):
            pltpu.async_copy(ms_bufs[b], accum.at[idx_bufs[b]], asems[b],
                             add=True)

        def a_wait(b):
            pltpu.make_async_copy(ms_bufs[b], accum.at[idx_bufs[b]],
                                  asems[b]).wait()

        for phase in range(2):
            q = c * 2 + phase
            pltpu.sync_copy(init_hbm.at[pl.ds(q * _N + s * _NROW, _NROW)],
                            accum.at[pl.ds(s * _NROW, _NROW)])

            @pl.when(s == _NS - 1)
            def _():
                pltpu.sync_copy(init_hbm.at[pl.ds(q * _N + rem0, _NREM)],
                                accum.at[pl.ds(rem0, _NREM)])

            plsc.subcore_barrier()

            for b in range(slots):
                l_start(q, b, b)

            def body(i2, carry):
                for b in range(slots):
                    chk = i2 * slots + b
                    l_wait(b)
                    a_start(b)
                    nxt = chk + slots

                    @pl.when(nxt < nch)
                    def _():
                        a_wait(b)
                        l_start(q, nxt, b)

                return carry

            lax.fori_loop(0, nit, body, 0)
            for t in range(full, nch):
                b = t % slots
                l_wait(b)
                a_start(b)
            for b in range(slots):
                a_wait(b)
            plsc.subcore_barrier()
            pltpu.sync_copy(accum.at[pl.ds(s * _NROW, _NROW)],
                            out_hbm.at[pl.ds(q * _N + s * _NROW, _NROW)])

            @pl.when(s == _NS - 1)
            def _():
                pltpu.sync_copy(accum.at[pl.ds(rem0, _NREM)],
                                out_hbm.at[pl.ds(q * _N + rem0, _NREM)])

            plsc.subcore_barrier()

    return k(ms_flat, dst, init)


def kernel(h, edge_index, e, emb_h, We, be, Wl, bl, bn, Ws, bs,
           W1, b1, W2, b2, W3, b3):
    src = [lax.slice_in_dim(edge_index[0], _EOFF[j], _EOFF[j] + _EH[j])
           for j in range(2)]
    dst = [lax.slice_in_dim(edge_index[1], _EOFF[j], _EOFF[j] + _EH[j])
           for j in range(2)]
    eraw = [lax.slice_in_dim(e, _EOFF[j], _EOFF[j] + _EH[j]) for j in range(2)]
    h_f = h.reshape(_N, 1)
    we2 = We.reshape(1, _H)
    be2 = be.reshape(1, _H)
    zeros4n = jnp.zeros((4 * _N, 128), _f32)

    hf = _embed(h_f, emb_h)
    eprev = [None, None]
    carry = [None, None]
    stats = [None, None]
    s0 = None
    for i in range(4):
        ah, bd_i, et_i = _nodemm(hf, Wl[i], bl[i])
        w2 = Wl[i, 2]
        b2e = bl[i, 2].reshape(1, _H)
        gath = [
            _gather_sc(bd_i, et_i, src[0], dst[0], _EH[0], 32, 4),
            _gather_sc(bd_i, et_i, src[1], dst[1], _EH[1], 32, 4),
        ]
        ms4 = [None, None]
        if i == 0:
            for j in range(2):
                carry[j], eprev[j], ms4[j], es, eq = _edge_first(
                    eraw[j], we2, be2, gath[j][0], gath[j][1], w2, b2e,
                    _EH[j])
                stats[j] = (es, eq)
        else:
            allstats = (stats[0][0], stats[1][0], stats[0][1], stats[1][1])
            if i < 3:
                nstats = [None, None]
                for j in range(2):
                    carry[j], eprev[j], ms4[j], es, eq = _edge_mid(
                        eprev[j], carry[j], gath[j][0], gath[j][1],
                        allstats, bn[i - 1], w2, b2e, _EH[j])
                    nstats[j] = (es, eq)
                stats = nstats
            else:
                for j in range(2):
                    ms4[j] = _edge_last(
                        eprev[j], carry[j], gath[j][0], gath[j][1],
                        allstats, bn[i - 1], w2, b2e, _EH[j])
        nd = _scatter_sc(ms4[0], dst[0], zeros4n, _EH[0], 80, 4)
        nd = _scatter_sc(ms4[1], dst[1], nd, _EH[1], 80, 4)
        hnew, hsum, hsq = _hnew(ah, nd.reshape(4, _N, 128))
        hf = _hout(hf, hnew, hsum, hsq, bn[i])
        if i == 2:
            s0 = _assign(hf, Ws, bs.reshape(1, _ASSIGN))

    h_out = _readout(hf, W1, b1.reshape(1, _H // 2),
                     W2, b2.reshape(1, _H // 4),
                     W3, b3.reshape(1, 8))
    return (h_out, s0.reshape(1, _N, _ASSIGN))


# speedup vs baseline: 4.4161x; 1.0508x over previous
import functools

import jax
import jax.numpy as jnp
from jax import lax
from jax.experimental import pallas as pl
from jax.experimental.pallas import tpu as pltpu
from jax.experimental.pallas import tpu_sc as plsc

_N = 10000
_E = 160000
_H = 256
_IN_DIM = 128
_ASSIGN = 64
_NB = 2000
_EB = 1280
_f32 = jnp.float32

_NC = 2
_NS = 16
_NW = _NC * _NS

_EH = (81920, 78080)
_EOFF = (0, 81920)


def _embed_body(h_ref, emb_ref, out_ref):
    hb = h_ref[...]
    io = lax.broadcasted_iota(jnp.int32, (_NB, _IN_DIM), 1)
    oh = (io == hb).astype(_f32)
    out_ref[...] = jnp.dot(oh, emb_ref[...], preferred_element_type=_f32)


def _embed(h_f, emb):
    return pl.pallas_call(
        _embed_body,
        grid=(_N // _NB,),
        in_specs=[
            pl.BlockSpec((_NB, 1), lambda i: (i, 0)),
            pl.BlockSpec((_IN_DIM, _H), lambda i: (0, 0)),
        ],
        out_specs=pl.BlockSpec((_NB, _H), lambda i: (i, 0)),
        out_shape=jax.ShapeDtypeStruct((_N, _H), _f32),
    )(h_f, emb)


_HI_MASK = -65536


def _pack2(lo, hi):
    lob = lax.bitcast_convert_type(lo, jnp.int32) + 0x8000
    hib = lax.bitcast_convert_type(hi, jnp.int32) + 0x8000
    return jnp.bitwise_or(jnp.bitwise_and(hib, _HI_MASK),
                          jnp.bitwise_and(jnp.right_shift(lob, 16), 0xFFFF))


def _unpack_lo(w):
    return lax.bitcast_convert_type(jnp.left_shift(w, 16), _f32)


def _unpack_hi(w):
    return lax.bitcast_convert_type(jnp.bitwise_and(w, _HI_MASK), _f32)


def _nodemm_body(hf_ref, w_ref, b_ref, ah_ref, bd_ref, et_ref):
    hf = hf_ref[...]
    w = w_ref[...]
    b = b_ref[...]
    ah_ref[...] = jnp.dot(hf, w[0], preferred_element_type=_f32) + b[0:1, :]
    bh = jnp.dot(hf, w[1], preferred_element_type=_f32) + b[1:2, :]
    dh = jnp.dot(hf, w[3], preferred_element_type=_f32) + b[3:4, :]
    bd_ref[...] = _pack2(bh, dh)
    eh = jnp.dot(hf, w[4], preferred_element_type=_f32) + b[4:5, :]
    et_ref[...] = _pack2(eh[:, :_H // 2], eh[:, _H // 2:])


def _nodemm(hf, w, b):
    return pl.pallas_call(
        _nodemm_body,
        grid=(_N // _NB,),
        in_specs=[
            pl.BlockSpec((_NB, _H), lambda i: (i, 0)),
            pl.BlockSpec((5, _H, _H), lambda i: (0, 0, 0)),
            pl.BlockSpec((5, _H), lambda i: (0, 0)),
        ],
        out_specs=[
            pl.BlockSpec((_NB, _H), lambda i: (i, 0)),
            pl.BlockSpec((_NB, _H), lambda i: (i, 0)),
            pl.BlockSpec((_NB, _H // 2), lambda i: (i, 0)),
        ],
        out_shape=[
            jax.ShapeDtypeStruct((_N, _H), _f32),
            jax.ShapeDtypeStruct((_N, _H), jnp.int32),
            jax.ShapeDtypeStruct((_N, _H // 2), jnp.int32),
        ],
    )(hf, w, b)


def _edge_core(e_in, gbd_w, ge_w, w2, b2, ms_o):
    ce = jnp.dot(e_in, w2, preferred_element_type=_f32) + b2
    ge = jnp.concatenate([_unpack_lo(ge_w), _unpack_hi(ge_w)], axis=1)
    ep = _unpack_hi(gbd_w) + ge + ce
    sig = jax.nn.sigmoid(ep)
    msg = sig * _unpack_lo(gbd_w)
    ms_o[0, :, :] = msg[:, :128]
    ms_o[1, :, :] = msg[:, 128:]
    ms_o[2, :, :] = sig[:, :128]
    ms_o[3, :, :] = sig[:, 128:]
    return ep


def _acc_stats(i, ep, esum_o, esq_o):
    @pl.when(i == 0)
    def _():
        esum_o[...] = jnp.zeros_like(esum_o)
        esq_o[...] = jnp.zeros_like(esq_o)

    esum_o[...] += jnp.sum(ep, axis=0, keepdims=True)
    esq_o[...] += jnp.sum(ep * ep, axis=0, keepdims=True)


def _edge_first_body(eraw_ref, we_ref, be_ref, gbd_ref, ge_ref, w2_ref, b2_ref,
                     carry_o, epre_o, ms_o, esum_o, esq_o):
    i = pl.program_id(0)
    e_in = eraw_ref[...] * we_ref[...] + be_ref[...]
    carry_o[...] = _pack2(e_in[:, :_H // 2], e_in[:, _H // 2:])
    ep = _edge_core(e_in, gbd_ref[...], ge_ref[...],
                    w2_ref[...], b2_ref[...], ms_o)
    epre_o[...] = _pack2(ep[:, :_H // 2], ep[:, _H // 2:])
    _acc_stats(i, ep, esum_o, esq_o)


def _bn_ein(eprev_ref, carry_ref, esa_ref, esb_ref, eqa_ref, eqb_ref, bnp_ref):
    esum = esa_ref[...] + esb_ref[...]
    esq = eqa_ref[...] + eqb_ref[...]
    mean = esum * (1.0 / _E)
    var = esq * (1.0 / _E) - mean * mean
    inv = lax.rsqrt(var + 1e-5)
    g = bnp_ref[2:3, :]
    bt = bnp_ref[3:4, :]
    ew = eprev_ref[...]
    ep = jnp.concatenate([_unpack_lo(ew), _unpack_hi(ew)], axis=1)
    cw = carry_ref[...]
    carry = jnp.concatenate([_unpack_lo(cw), _unpack_hi(cw)], axis=1)
    return carry + jnp.maximum(g * (ep - mean) * inv + bt, 0.0)


def _edge_mid_body(eprev_ref, carry_ref, gbd_ref, ge_ref,
                   esa_ref, esb_ref, eqa_ref, eqb_ref,
                   bnp_ref, w2_ref, b2_ref,
                   carry_o, epre_o, ms_o, esum_o, esq_o):
    i = pl.program_id(0)
    e_in = _bn_ein(eprev_ref, carry_ref, esa_ref, esb_ref, eqa_ref, eqb_ref,
                   bnp_ref)
    carry_o[...] = _pack2(e_in[:, :_H // 2], e_in[:, _H // 2:])
    ep = _edge_core(e_in, gbd_ref[...], ge_ref[...],
                    w2_ref[...], b2_ref[...], ms_o)
    epre_o[...] = _pack2(ep[:, :_H // 2], ep[:, _H // 2:])
    _acc_stats(i, ep, esum_o, esq_o)


def _edge_last_body(eprev_ref, carry_ref, gbd_ref, ge_ref,
                    esa_ref, esb_ref, eqa_ref, eqb_ref,
                    bnp_ref, w2_ref, b2_ref, ms_o):
    e_in = _bn_ein(eprev_ref, carry_ref, esa_ref, esb_ref, eqa_ref, eqb_ref,
                   bnp_ref)
    _edge_core(e_in, gbd_ref[...], ge_ref[...],
               w2_ref[...], b2_ref[...], ms_o)


_stat_spec = pl.BlockSpec((1, _H), lambda i: (0, 0))
_stat_shape = jax.ShapeDtypeStruct((1, _H), _f32)
_w2_spec = pl.BlockSpec((_H, _H), lambda i: (0, 0))
_erow_spec = pl.BlockSpec((_EB, _H), lambda i: (i, 0))
_ehw_spec = pl.BlockSpec((_EB, _H // 2), lambda i: (i, 0))
_ms4_spec = pl.BlockSpec((4, _EB, 128), lambda i: (0, i, 0))


def _edge_first(eraw, we, be, gbd, ge, w2, b2, eh):
    return pl.pallas_call(
        _edge_first_body,
        grid=(eh // _EB,),
        in_specs=[
            pl.BlockSpec((_EB, 1), lambda i: (i, 0)),
            _stat_spec, _stat_spec,
            _erow_spec,
            _ehw_spec,
            _w2_spec, _stat_spec,
        ],
        out_specs=[_ehw_spec, _ehw_spec, _ms4_spec, _stat_spec, _stat_spec],
        out_shape=[
            jax.ShapeDtypeStruct((eh, _H // 2), jnp.int32),
            jax.ShapeDtypeStruct((eh, _H // 2), jnp.int32),
            jax.ShapeDtypeStruct((4, eh, 128), _f32),
            _stat_shape, _stat_shape,
        ],
    )(eraw, we, be, gbd, ge, w2, b2)


def _edge_mid(eprev, carry, gbd, ge, stats, bnp, w2, b2, eh):
    return pl.pallas_call(
        _edge_mid_body,
        grid=(eh // _EB,),
        in_specs=[
            _ehw_spec, _ehw_spec,
            _erow_spec,
            _ehw_spec,
            _stat_spec, _stat_spec, _stat_spec, _stat_spec,
            pl.BlockSpec((4, _H), lambda i: (0, 0)),
            _w2_spec, _stat_spec,
        ],
        out_specs=[_ehw_spec, _ehw_spec, _ms4_spec, _stat_spec, _stat_spec],
        out_shape=[
            jax.ShapeDtypeStruct((eh, _H // 2), jnp.int32),
            jax.ShapeDtypeStruct((eh, _H // 2), jnp.int32),
            jax.ShapeDtypeStruct((4, eh, 128), _f32),
            _stat_shape, _stat_shape,
        ],
    )(eprev, carry, gbd, ge, *stats, bnp, w2, b2)


def _edge_last(eprev, carry, gbd, ge, stats, bnp, w2, b2, eh):
    return pl.pallas_call(
        _edge_last_body,
        grid=(eh // _EB,),
        in_specs=[
            _ehw_spec, _ehw_spec,
            _erow_spec,
            _ehw_spec,
            _stat_spec, _stat_spec, _stat_spec, _stat_spec,
            pl.BlockSpec((4, _H), lambda i: (0, 0)),
            _w2_spec, _stat_spec,
        ],
        out_specs=_ms4_spec,
        out_shape=jax.ShapeDtypeStruct((4, eh, 128), _f32),
    )(eprev, carry, gbd, ge, *stats, bnp, w2, b2)


def _hnew_body(ah_ref, nd_ref, hnew_o, hsum_o, hsq_o):
    i = pl.program_id(0)
    nd = nd_ref[...]
    num = jnp.concatenate([nd[0], nd[1]], axis=1)
    den = jnp.concatenate([nd[2], nd[3]], axis=1)
    hn = ah_ref[...] + num / (den + 1e-6)
    hnew_o[...] = hn
    _acc_stats(i, hn, hsum_o, hsq_o)


def _hnew(ah, nd):
    return pl.pallas_call(
        _hnew_body,
        grid=(_N // _NB,),
        in_specs=[
            pl.BlockSpec((_NB, _H), lambda i: (i, 0)),
            pl.BlockSpec((4, _NB, 128), lambda i: (0, i, 0)),
        ],
        out_specs=[pl.BlockSpec((_NB, _H), lambda i: (i, 0)),
                   _stat_spec, _stat_spec],
        out_shape=[jax.ShapeDtypeStruct((_N, _H), _f32),
                   _stat_shape, _stat_shape],
    )(ah, nd)


def _hout_body(hin_ref, hnew_ref, hsum_ref, hsq_ref, bnp_ref, out_o):
    mean = hsum_ref[...] * (1.0 / _N)
    var = hsq_ref[...] * (1.0 / _N) - mean * mean
    inv = lax.rsqrt(var + 1e-5)
    g = bnp_ref[0:1, :]
    bt = bnp_ref[1:2, :]
    out_o[...] = hin_ref[...] + jnp.maximum(
        g * (hnew_ref[...] - mean) * inv + bt, 0.0)


def _hout(hin, hnew, hsum, hsq, bnp):
    return pl.pallas_call(
        _hout_body,
        grid=(_N // _NB,),
        in_specs=[
            pl.BlockSpec((_NB, _H), lambda i: (i, 0)),
            pl.BlockSpec((_NB, _H), lambda i: (i, 0)),
            _stat_spec, _stat_spec,
            pl.BlockSpec((4, _H), lambda i: (0, 0)),
        ],
        out_specs=pl.BlockSpec((_NB, _H), lambda i: (i, 0)),
        out_shape=jax.ShapeDtypeStruct((_N, _H), _f32),
    )(hin, hnew, hsum, hsq, bnp)


def _assign_body(hf_ref, ws_ref, bs_ref, out_o):
    lg = jnp.dot(hf_ref[...], ws_ref[...], preferred_element_type=_f32) + bs_ref[...]
    m = jnp.max(lg, axis=1, keepdims=True)
    ex = jnp.exp(lg - m)
    out_o[...] = ex / jnp.sum(ex, axis=1, keepdims=True)


def _assign(hf, ws, bs):
    return pl.pallas_call(
        _assign_body,
        grid=(_N // _NB,),
        in_specs=[
            pl.BlockSpec((_NB, _H), lambda i: (i, 0)),
            pl.BlockSpec((_H, _ASSIGN), lambda i: (0, 0)),
            pl.BlockSpec((1, _ASSIGN), lambda i: (0, 0)),
        ],
        out_specs=pl.BlockSpec((_NB, _ASSIGN), lambda i: (i, 0)),
        out_shape=jax.ShapeDtypeStruct((_N, _ASSIGN), _f32),
    )(hf, ws, bs)


def _readout_body(hf_ref, w1_ref, b1_ref, w2_ref, b2_ref, w3_ref, b3_ref, out_o):
    x = jnp.maximum(
        jnp.dot(hf_ref[...], w1_ref[...], preferred_element_type=_f32) + b1_ref[...], 0.0)
    x = jnp.maximum(
        jnp.dot(x, w2_ref[...], preferred_element_type=_f32) + b2_ref[...], 0.0)
    out_o[...] = jnp.dot(x, w3_ref[...], preferred_element_type=_f32) + b3_ref[...]


def _readout(hf, w1, b1, w2, b2, w3, b3):
    return pl.pallas_call(
        _readout_body,
        grid=(_N // _NB,),
        in_specs=[
            pl.BlockSpec((_NB, _H), lambda i: (i, 0)),
            pl.BlockSpec((_H, _H // 2), lambda i: (0, 0)),
            pl.BlockSpec((1, _H // 2), lambda i: (0, 0)),
            pl.BlockSpec((_H // 2, _H // 4), lambda i: (0, 0)),
            pl.BlockSpec((1, _H // 4), lambda i: (0, 0)),
            pl.BlockSpec((_H // 4, 8), lambda i: (0, 0)),
            pl.BlockSpec((1, 8), lambda i: (0, 0)),
        ],
        out_specs=pl.BlockSpec((_NB, 8), lambda i: (i, 0)),
        out_shape=jax.ShapeDtypeStruct((_N, 8), _f32),
    )(hf, w1, b1, w2, b2, w3, b3)


_NROW = 624
_NREM = _N - _NS * _NROW


def _mesh():
    return plsc.VectorSubcoreMesh(
        core_axis_name="c", subcore_axis_name="s",
        num_cores=_NC, num_subcores=_NS)


def _gather_sc(bd, et, src, dst, eh, ch, slots):
    wbd = _H
    wet = _H // 2
    per = eh // _NW
    nch = per // ch
    rem = per - nch * ch
    full = (nch // slots) * slots
    nit = full // slots

    @functools.partial(
        pl.kernel,
        out_type=[
            jax.ShapeDtypeStruct((eh, wbd), jnp.int32),
            jax.ShapeDtypeStruct((eh, wet), jnp.int32),
        ],
        mesh=_mesh(),
        scratch_types=[
            pltpu.VMEM((per,), jnp.int32),
            pltpu.VMEM((per,), jnp.int32),
            [pltpu.VMEM((ch, wbd), jnp.int32) for _ in range(slots)],
            [pltpu.VMEM((ch, wet), jnp.int32) for _ in range(slots)],
            [pltpu.SemaphoreType.DMA for _ in range(slots)],
            [pltpu.SemaphoreType.DMA for _ in range(slots)],
        ],
    )
    def k(bd_hbm, et_hbm, src_hbm, dst_hbm, gbd_hbm, ge_hbm,
          idx_s, idx_d, bd_bufs, e_bufs, gsems, wsems):
        c = lax.axis_index("c")
        s = lax.axis_index("s")
        wid = s * _NC + c
        start = wid * per
        pltpu.sync_copy(src_hbm.at[pl.ds(start, per)], idx_s)
        pltpu.sync_copy(dst_hbm.at[pl.ds(start, per)], idx_d)

        def g_start(chk, b):
            off = chk * ch
            pltpu.async_copy(bd_hbm.at[idx_s.at[pl.ds(off, ch)]],
                             bd_bufs[b], gsems[b])
            pltpu.async_copy(et_hbm.at[idx_d.at[pl.ds(off, ch)]],
                             e_bufs[b], gsems[b])

        def g_wait(b):
            pltpu.make_async_copy(bd_hbm.at[idx_s.at[pl.ds(0, ch)]],
                                  bd_bufs[b], gsems[b]).wait()
            pltpu.make_async_copy(et_hbm.at[idx_d.at[pl.ds(0, ch)]],
                                  e_bufs[b], gsems[b]).wait()

        def w_start(chk, b):
            off = start + chk * ch
            pltpu.async_copy(bd_bufs[b], gbd_hbm.at[pl.ds(off, ch)], wsems[b])
            pltpu.async_copy(e_bufs[b], ge_hbm.at[pl.ds(off, ch)], wsems[b])

        def w_wait(b):
            pltpu.make_async_copy(bd_bufs[b], gbd_hbm.at[pl.ds(0, ch)],
                                  wsems[b]).wait()
            pltpu.make_async_copy(e_bufs[b], ge_hbm.at[pl.ds(0, ch)],
                                  wsems[b]).wait()

        for b in range(slots):
            g_start(b, b)

        def body(i2, carry):
            for b in range(slots):
                chk = i2 * slots + b
                g_wait(b)
                w_start(chk, b)
                nxt = chk + slots

                @pl.when(nxt < nch)
                def _():
                    w_wait(b)
                    g_start(nxt, b)

            return carry

        lax.fori_loop(0, nit, body, 0)
        for t in range(full, nch):
            b = t % slots
            g_wait(b)
            w_start(t, b)
        for b in range(slots):
            w_wait(b)
        if rem:
            toff = nch * ch
            base = start + toff
            pltpu.sync_copy(bd_hbm.at[idx_s.at[pl.ds(toff, rem)]],
                            bd_bufs[0].at[pl.ds(0, rem)])
            pltpu.sync_copy(et_hbm.at[idx_d.at[pl.ds(toff, rem)]],
                            e_bufs[0].at[pl.ds(0, rem)])
            pltpu.sync_copy(bd_bufs[0].at[pl.ds(0, rem)],
                            gbd_hbm.at[pl.ds(base, rem)])
            pltpu.sync_copy(e_bufs[0].at[pl.ds(0, rem)],
                            ge_hbm.at[pl.ds(base, rem)])

    return k(bd, et, src, dst)


def _scatter_sc(ms4, dst, init, eh, ch, slots):
    ms_flat = ms4.reshape(4 * eh, 128)
    per = eh // _NS
    nch = per // ch
    full = (nch // slots) * slots
    nit = full // slots

    @functools.partial(
        pl.kernel,
        out_type=jax.ShapeDtypeStruct((4 * _N, 128), _f32),
        mesh=_mesh(),
        scratch_types=[
            pltpu.VMEM_SHARED((_N, 128), _f32),
            [pltpu.VMEM((ch,), jnp.int32) for _ in range(slots)],
            [pltpu.VMEM((ch, 128), _f32) for _ in range(slots)],
            [pltpu.SemaphoreType.DMA for _ in range(slots)],
            [pltpu.SemaphoreType.DMA for _ in range(slots)],
        ],
    )
    def k(ms_hbm, dst_hbm, init_hbm, out_hbm, accum, idx_bufs, ms_bufs,
          lsems, asems):
        c = lax.axis_index("c")
        s = lax.axis_index("s")
        rem0 = _NS * _NROW

        def l_start(q, chk, b):
            base = s * per + chk * ch
            pltpu.async_copy(dst_hbm.at[pl.ds(base, ch)], idx_bufs[b],
                             lsems[b])
            pltpu.async_copy(ms_hbm.at[pl.ds(q * eh + base, ch)],
                             ms_bufs[b], lsems[b])

        def l_wait(b):
            pltpu.make_async_copy(dst_hbm.at[pl.ds(0, ch)], idx_bufs[b],
                                  lsems[b]).wait()
            pltpu.make_async_copy(ms_hbm.at[pl.ds(0, ch)], ms_bufs[b],
                                  lsems[b]).wait()

        def a_start(b):
            pltpu.async_copy(ms_bufs[b], accum.at[idx_bufs[b]], asems[b],
                             add=True)

        def a_wait(b):
            pltpu.make_async_copy(ms_bufs[b], accum.at[idx_bufs[b]],
                                  asems[b]).wait()

        for phase in range(2):
            q = c * 2 + phase
            pltpu.sync_copy(init_hbm.at[pl.ds(q * _N + s * _NROW, _NROW)],
                            accum.at[pl.ds(s * _NROW, _NROW)])

            @pl.when(s == _NS - 1)
            def _():
                pltpu.sync_copy(init_hbm.at[pl.ds(q * _N + rem0, _NREM)],
                                accum.at[pl.ds(rem0, _NREM)])

            plsc.subcore_barrier()

            for b in range(slots):
                l_start(q, b, b)

            def body(i2, carry):
                for b in range(slots):
                    chk = i2 * slots + b
                    l_wait(b)
                    a_start(b)
                    nxt = chk + slots

                    @pl.when(nxt < nch)
                    def _():
                        a_wait(b)
                        l_start(q, nxt, b)

                return carry

            lax.fori_loop(0, nit, body, 0)
            for t in range(full, nch):
                b = t % slots
                l_wait(b)
                a_start(b)
            for b in range(slots):
                a_wait(b)
            plsc.subcore_barrier()
            pltpu.sync_copy(accum.at[pl.ds(s * _NROW, _NROW)],
                            out_hbm.at[pl.ds(q * _N + s * _NROW, _NROW)])

            @pl.when(s == _NS - 1)
            def _():
                pltpu.sync_copy(accum.at[pl.ds(rem0, _NREM)],
                                out_hbm.at[pl.ds(q * _N + rem0, _NREM)])

            plsc.subcore_barrier()

    return k(ms_flat, dst, init)


def kernel(h, edge_index, e, emb_h, We, be, Wl, bl, bn, Ws, bs,
           W1, b1, W2, b2, W3, b3):
    src = [lax.slice_in_dim(edge_index[0], _EOFF[j], _EOFF[j] + _EH[j])
           for j in range(2)]
    dst = [lax.slice_in_dim(edge_index[1], _EOFF[j], _EOFF[j] + _EH[j])
           for j in range(2)]
    eraw = [lax.slice_in_dim(e, _EOFF[j], _EOFF[j] + _EH[j]) for j in range(2)]
    h_f = h.reshape(_N, 1)
    we2 = We.reshape(1, _H)
    be2 = be.reshape(1, _H)
    zeros4n = jnp.zeros((4 * _N, 128), _f32)

    hf = _embed(h_f, emb_h)
    eprev = [None, None]
    carry = [None, None]
    stats = [None, None]
    s0 = None
    for i in range(4):
        ah, bd_i, et_i = _nodemm(hf, Wl[i], bl[i])
        w2 = Wl[i, 2]
        b2e = bl[i, 2].reshape(1, _H)
        gath = [
            _gather_sc(bd_i, et_i, src[0], dst[0], _EH[0], 32, 4),
            _gather_sc(bd_i, et_i, src[1], dst[1], _EH[1], 32, 4),
        ]
        ms4 = [None, None]
        if i == 0:
            for j in range(2):
                carry[j], eprev[j], ms4[j], es, eq = _edge_first(
                    eraw[j], we2, be2, gath[j][0], gath[j][1], w2, b2e,
                    _EH[j])
                stats[j] = (es, eq)
        else:
            allstats = (stats[0][0], stats[1][0], stats[0][1], stats[1][1])
            if i < 3:
                nstats = [None, None]
                for j in range(2):
                    carry[j], eprev[j], ms4[j], es, eq = _edge_mid(
                        eprev[j], carry[j], gath[j][0], gath[j][1],
                        allstats, bn[i - 1], w2, b2e, _EH[j])
                    nstats[j] = (es, eq)
                stats = nstats
            else:
                for j in range(2):
                    ms4[j] = _edge_last(
                        eprev[j], carry[j], gath[j][0], gath[j][1],
                        allstats, bn[i - 1], w2, b2e, _EH[j])
        nd = _scatter_sc(ms4[0], dst[0], zeros4n, _EH[0], 80, 4)
        nd = _scatter_sc(ms4[1], dst[1], nd, _EH[1], 80, 4)
        hnew, hsum, hsq = _hnew(ah, nd.reshape(4, _N, 128))
        hf = _hout(hf, hnew, hsum, hsq, bn[i])
        if i == 2:
            s0 = _assign(hf, Ws, bs.reshape(1, _ASSIGN))

    h_out = _readout(hf, W1, b1.reshape(1, _H // 2),
                     W2, b2.reshape(1, _H // 4),
                     W3, b3.reshape(1, 8))
    return (h_out, s0.reshape(1, _N, _ASSIGN))
